# Initial kernel scaffold; baseline (speedup 1.0000x reference)
import jax
import jax.numpy as jnp
from jax.experimental import pallas as pl


def kernel(x, edge_index, W1, b1, a1, W2, b2):
    raise NotImplementedError("write your pallas kernel here")


# speedup vs baseline: 10.6231x; 10.6231x over previous
import functools
import jax
import jax.numpy as jnp
from jax import lax
from jax.experimental import pallas as pl
from jax.experimental.pallas import tpu as pltpu
from jax.experimental.pallas import tpu_sc as plsc

NN = 10000
EE = 320000
DD = 128
NC = 2
NS = 16
NW = NC * NS
EPW = EE // NW
CH = 128
EPWP = 10240
NCHUNK = EPWP // CH
GRP = 8
NGRP = NCHUNK // GRP
NNP = 10240
RPT = NNP // NS
TRASH = 10100

_mesh = plsc.VectorSubcoreMesh(
    core_axis_name="c", subcore_axis_name="s", num_cores=NC, num_subcores=NS
)


def _deg_body(col_hbm, degp_hbm, colv, hist, sem):
    c = lax.axis_index("c")
    s = lax.axis_index("s")
    wid = c * NS + s
    pltpu.async_copy(col_hbm.at[wid], colv, sem).wait()
    zeros16 = jnp.zeros((16,), jnp.float32)

    def zero_step(i, carry):
        hist[pl.ds(i * 16, 16)] = zeros16
        return carry

    lax.fori_loop(0, NN // 16, zero_step, 0)
    ones16 = jnp.ones((16,), jnp.float32)

    def acc_step(i, carry):
        idx = colv[pl.ds(i * 16, 16)]
        plsc.addupdate_scatter(hist, [idx], ones16)
        return carry

    lax.fori_loop(0, EPW // 16, acc_step, 0)
    pltpu.sync_copy(hist, degp_hbm.at[wid])


_deg_call = pl.kernel(
    _deg_body,
    out_type=jax.ShapeDtypeStruct((NW, NN), jnp.float32),
    mesh=_mesh,
    compiler_params=pltpu.CompilerParams(needs_layout_passes=False),
    scratch_types=[
        pltpu.VMEM((EPW,), jnp.int32),
        pltpu.VMEM((NN,), jnp.float32),
        pltpu.SemaphoreType.DMA,
    ],
)


def _agg_body(hp_hbm, row3_hbm, col3_hbm, zeros_hbm, acc_hbm,
              rowv, colv, g0, g1, acc_sh, sem):
    c = lax.axis_index("c")
    s = lax.axis_index("s")
    wid = c * NS + s
    pltpu.sync_copy(zeros_hbm, acc_sh.at[pl.ds(s * RPT, RPT)])
    plsc.subcore_barrier()

    gbufs = (g0, g1)

    def group(gi, carry):
        pltpu.async_copy(row3_hbm.at[wid].at[pl.ds(gi * GRP, GRP)], rowv, sem).wait()
        pltpu.async_copy(col3_hbm.at[wid].at[pl.ds(gi * GRP, GRP)], colv, sem).wait()
        pltpu.async_copy(hp_hbm.at[rowv.at[0]], g0, sem)
        for b in range(GRP - 1):
            cur = gbufs[b % 2]
            nxt = gbufs[(b + 1) % 2]
            pltpu.make_async_copy(hp_hbm.at[rowv.at[b]], cur, sem).wait()
            pltpu.async_copy(hp_hbm.at[rowv.at[b + 1]], nxt, sem)
            pltpu.sync_copy(cur, acc_sh.at[colv.at[b]], add=True)
        last = gbufs[(GRP - 1) % 2]
        pltpu.make_async_copy(hp_hbm.at[rowv.at[GRP - 1]], last, sem).wait()
        pltpu.sync_copy(last, acc_sh.at[colv.at[GRP - 1]], add=True)
        return carry

    lax.fori_loop(0, NGRP, group, 0)

    plsc.subcore_barrier()
    pltpu.sync_copy(
        acc_sh.at[pl.ds(s * RPT, RPT)],
        acc_hbm.at[c].at[pl.ds(s * RPT, RPT)],
    )


_agg_call = pl.kernel(
    _agg_body,
    out_type=jax.ShapeDtypeStruct((NC, NNP, DD), jnp.float32),
    mesh=_mesh,
    scratch_types=[
        pltpu.VMEM((GRP, CH), jnp.int32),
        pltpu.VMEM((GRP, CH), jnp.int32),
        pltpu.VMEM((CH, DD), jnp.float32),
        pltpu.VMEM((CH, DD), jnp.float32),
        pltpu.VMEM_SHARED((NNP, DD), jnp.float32),
        pltpu.SemaphoreType.DMA,
    ],
)


BR = 2000


def _kd_body(degp_ref, out_ref):
    deg = jnp.sum(degp_ref[...], axis=0) + 1.0
    out_ref[...] = lax.rsqrt(deg)[:, None]


_kd_call = pl.pallas_call(
    _kd_body,
    in_specs=[pl.BlockSpec((NW, NN), lambda: (0, 0))],
    out_specs=pl.BlockSpec((NN, 1), lambda: (0, 0)),
    out_shape=jax.ShapeDtypeStruct((NN, 1), jnp.float32),
)


def _k2_body(x_ref, w_ref, dinv_ref, out_ref):
    h = jnp.dot(x_ref[...], w_ref[...], preferred_element_type=jnp.float32,
                precision=lax.Precision.HIGHEST)
    out_ref[...] = h * dinv_ref[...]


_k2_call = pl.pallas_call(
    _k2_body,
    grid=(NN // BR,),
    in_specs=[
        pl.BlockSpec((BR, DD), lambda i: (i, 0)),
        pl.BlockSpec((DD, DD), lambda i: (0, 0)),
        pl.BlockSpec((BR, 1), lambda i: (i, 0)),
    ],
    out_specs=pl.BlockSpec((BR, DD), lambda i: (i, 0)),
    out_shape=jax.ShapeDtypeStruct((NN, DD), jnp.float32),
)


def _k4_body(acc_ref, hp_ref, dinv_ref, w_ref, b_ref, a_ref, out_ref):
    dinv = dinv_ref[...]
    tot = acc_ref[0] + acc_ref[1] + hp_ref[...]
    h = tot * dinv + b_ref[...]
    h = jnp.where(h >= 0, h, a_ref[0, 0] * h)
    out_ref[...] = jnp.dot(h, w_ref[...], preferred_element_type=jnp.float32,
                           precision=lax.Precision.HIGHEST) * dinv


_k4_call = pl.pallas_call(
    _k4_body,
    grid=(NN // BR,),
    in_specs=[
        pl.BlockSpec((NC, BR, DD), lambda i: (0, i, 0)),
        pl.BlockSpec((BR, DD), lambda i: (i, 0)),
        pl.BlockSpec((BR, 1), lambda i: (i, 0)),
        pl.BlockSpec((DD, DD), lambda i: (0, 0)),
        pl.BlockSpec((1, DD), lambda i: (0, 0)),
        pl.BlockSpec((1, 1), lambda i: (0, 0), memory_space=pltpu.SMEM),
    ],
    out_specs=pl.BlockSpec((BR, DD), lambda i: (i, 0)),
    out_shape=jax.ShapeDtypeStruct((NN, DD), jnp.float32),
)


def _k6_body(acc_ref, hp_ref, dinv_ref, b_ref, out_ref):
    tot = acc_ref[0] + acc_ref[1] + hp_ref[...]
    out_ref[...] = tot * dinv_ref[...] + b_ref[...]


_k6_call = pl.pallas_call(
    _k6_body,
    grid=(NN // BR,),
    in_specs=[
        pl.BlockSpec((NC, BR, DD), lambda i: (0, i, 0)),
        pl.BlockSpec((BR, DD), lambda i: (i, 0)),
        pl.BlockSpec((BR, 1), lambda i: (i, 0)),
        pl.BlockSpec((1, DD), lambda i: (0, 0)),
    ],
    out_specs=pl.BlockSpec((BR, DD), lambda i: (i, 0)),
    out_shape=jax.ShapeDtypeStruct((NN, DD), jnp.float32),
)


@jax.jit
def kernel(x, edge_index, W1, b1, a1, W2, b2):
    row = edge_index[0].astype(jnp.int32)
    col = edge_index[1].astype(jnp.int32)
    cole = col.reshape(NW, EPW)
    pad = ((0, 0), (0, EPWP - EPW))
    row2 = jnp.pad(row.reshape(NW, EPW), pad).reshape(NW, NCHUNK, CH)
    col2 = jnp.pad(col.reshape(NW, EPW), pad,
                   constant_values=TRASH).reshape(NW, NCHUNK, CH)
    zeros_blk = jnp.zeros((RPT, DD), jnp.float32)
    b1r = b1.reshape(1, DD)
    b2r = b2.reshape(1, DD)
    a1r = a1.reshape(1, 1)

    degp = _deg_call(cole)
    dinv = _kd_call(degp)
    h1p = _k2_call(x, W1, dinv)
    acc1 = _agg_call(h1p, row2, col2, zeros_blk)
    h2p = _k4_call(acc1, h1p, dinv, W2, b1r, a1r)
    acc2 = _agg_call(h2p, row2, col2, zeros_blk)
    out = _k6_call(acc2, h2p, dinv, b2r)
    return out


# speedup vs baseline: 10.9196x; 1.0279x over previous
import functools
import jax
import jax.numpy as jnp
from jax import lax
from jax.experimental import pallas as pl
from jax.experimental.pallas import tpu as pltpu
from jax.experimental.pallas import tpu_sc as plsc

NN = 10000
EE = 320000
DD = 128
NC = 2
NS = 16
NW = NC * NS
EPW = EE // NW
CH = 128
EPWP = 10240
NCHUNK = EPWP // CH
GRP = 8
NGRP = NCHUNK // GRP
NNP = 10240
RPT = NNP // NS
TRASH = 10100

_mesh = plsc.VectorSubcoreMesh(
    core_axis_name="c", subcore_axis_name="s", num_cores=NC, num_subcores=NS
)


def _deg_body(col_hbm, degp_hbm, colv, hist, sem):
    c = lax.axis_index("c")
    s = lax.axis_index("s")
    wid = c * NS + s
    pltpu.async_copy(col_hbm.at[wid], colv, sem).wait()
    zeros16 = jnp.zeros((16,), jnp.float32)

    def zero_step(i, carry):
        hist[pl.ds(i * 16, 16)] = zeros16
        return carry

    lax.fori_loop(0, NN // 16, zero_step, 0)
    ones16 = jnp.ones((16,), jnp.float32)

    def acc_step(i, carry):
        idx = colv[pl.ds(i * 16, 16)]
        plsc.addupdate_scatter(hist, [idx], ones16)
        return carry

    lax.fori_loop(0, EPW // 16, acc_step, 0)
    pltpu.sync_copy(hist, degp_hbm.at[wid])


_deg_call = pl.kernel(
    _deg_body,
    out_type=jax.ShapeDtypeStruct((NW, NN), jnp.float32),
    mesh=_mesh,
    compiler_params=pltpu.CompilerParams(needs_layout_passes=False),
    scratch_types=[
        pltpu.VMEM((EPW,), jnp.int32),
        pltpu.VMEM((NN,), jnp.float32),
        pltpu.SemaphoreType.DMA,
    ],
)


def _agg_body(hp_hbm, row3_hbm, col3_hbm, zeros_hbm, acc_hbm,
              ir0, ic0, ir1, ic1, g0, g1, acc_sh, semi, semg, sems):
    c = lax.axis_index("c")
    s = lax.axis_index("s")
    wid = c * NS + s
    pltpu.sync_copy(zeros_hbm, acc_sh.at[pl.ds(s * RPT, RPT)])
    plsc.subcore_barrier()

    irows = (ir0, ir1)
    icols = (ic0, ic1)
    gbufs = (g0, g1)

    def slab(kind, gi, buf, start=False):
        src = (row3_hbm if kind == 0 else col3_hbm).at[wid].at[pl.ds(gi * GRP, GRP)]
        if start:
            pltpu.async_copy(src, buf, semi)
        else:
            pltpu.make_async_copy(src, buf, semi).wait()

    def gather(t, start=False):
        gi, b = divmod(t, GRP)
        src = hp_hbm.at[irows[gi % 2].at[b]]
        if start:
            pltpu.async_copy(src, gbufs[t % 2], semg)
        else:
            pltpu.make_async_copy(src, gbufs[t % 2], semg).wait()

    def scatter(t, start=False):
        gi, b = divmod(t, GRP)
        dst = acc_sh.at[icols[gi % 2].at[b]]
        if start:
            pltpu.async_copy(gbufs[t % 2], dst, sems, add=True)
        else:
            pltpu.make_async_copy(gbufs[t % 2], dst, sems).wait()

    slab(0, 0, ir0, start=True)
    slab(1, 0, ic0, start=True)
    slab(0, 0, ir0)
    slab(1, 0, ic0)
    gather(0, start=True)
    for t in range(NCHUNK):
        gi, b = divmod(t, GRP)
        gather(t)
        scatter(t, start=True)
        if t >= 1:
            scatter(t - 1)
        if b == 0 and gi + 1 < NGRP:
            slab(0, gi + 1, irows[(gi + 1) % 2], start=True)
            slab(1, gi + 1, icols[(gi + 1) % 2], start=True)
        if t + 1 < NCHUNK:
            gn, bn = divmod(t + 1, GRP)
            if bn == 0:
                slab(0, gn, irows[gn % 2])
                slab(1, gn, icols[gn % 2])
            gather(t + 1, start=True)
    scatter(NCHUNK - 1)

    plsc.subcore_barrier()
    pltpu.sync_copy(
        acc_sh.at[pl.ds(s * RPT, RPT)],
        acc_hbm.at[c].at[pl.ds(s * RPT, RPT)],
    )


_agg_call = pl.kernel(
    _agg_body,
    out_type=jax.ShapeDtypeStruct((NC, NNP, DD), jnp.float32),
    mesh=_mesh,
    scratch_types=[
        pltpu.VMEM((GRP, CH), jnp.int32),
        pltpu.VMEM((GRP, CH), jnp.int32),
        pltpu.VMEM((GRP, CH), jnp.int32),
        pltpu.VMEM((GRP, CH), jnp.int32),
        pltpu.VMEM((CH, DD), jnp.float32),
        pltpu.VMEM((CH, DD), jnp.float32),
        pltpu.VMEM_SHARED((NNP, DD), jnp.float32),
        pltpu.SemaphoreType.DMA,
        pltpu.SemaphoreType.DMA,
        pltpu.SemaphoreType.DMA,
    ],
)


BR = 2000


def _kd_body(degp_ref, out_ref):
    deg = jnp.sum(degp_ref[...], axis=0) + 1.0
    out_ref[...] = lax.rsqrt(deg)[:, None]


_kd_call = pl.pallas_call(
    _kd_body,
    in_specs=[pl.BlockSpec((NW, NN), lambda: (0, 0))],
    out_specs=pl.BlockSpec((NN, 1), lambda: (0, 0)),
    out_shape=jax.ShapeDtypeStruct((NN, 1), jnp.float32),
)


def _k2_body(x_ref, w_ref, dinv_ref, out_ref):
    h = jnp.dot(x_ref[...], w_ref[...], preferred_element_type=jnp.float32,
                precision=lax.Precision.HIGHEST)
    out_ref[...] = h * dinv_ref[...]


_k2_call = pl.pallas_call(
    _k2_body,
    grid=(NN // BR,),
    in_specs=[
        pl.BlockSpec((BR, DD), lambda i: (i, 0)),
        pl.BlockSpec((DD, DD), lambda i: (0, 0)),
        pl.BlockSpec((BR, 1), lambda i: (i, 0)),
    ],
    out_specs=pl.BlockSpec((BR, DD), lambda i: (i, 0)),
    out_shape=jax.ShapeDtypeStruct((NN, DD), jnp.float32),
)


def _k4_body(acc_ref, hp_ref, dinv_ref, w_ref, b_ref, a_ref, out_ref):
    dinv = dinv_ref[...]
    tot = acc_ref[0] + acc_ref[1] + hp_ref[...]
    h = tot * dinv + b_ref[...]
    h = jnp.where(h >= 0, h, a_ref[0, 0] * h)
    out_ref[...] = jnp.dot(h, w_ref[...], preferred_element_type=jnp.float32,
                           precision=lax.Precision.HIGHEST) * dinv


_k4_call = pl.pallas_call(
    _k4_body,
    grid=(NN // BR,),
    in_specs=[
        pl.BlockSpec((NC, BR, DD), lambda i: (0, i, 0)),
        pl.BlockSpec((BR, DD), lambda i: (i, 0)),
        pl.BlockSpec((BR, 1), lambda i: (i, 0)),
        pl.BlockSpec((DD, DD), lambda i: (0, 0)),
        pl.BlockSpec((1, DD), lambda i: (0, 0)),
        pl.BlockSpec((1, 1), lambda i: (0, 0), memory_space=pltpu.SMEM),
    ],
    out_specs=pl.BlockSpec((BR, DD), lambda i: (i, 0)),
    out_shape=jax.ShapeDtypeStruct((NN, DD), jnp.float32),
)


def _k6_body(acc_ref, hp_ref, dinv_ref, b_ref, out_ref):
    tot = acc_ref[0] + acc_ref[1] + hp_ref[...]
    out_ref[...] = tot * dinv_ref[...] + b_ref[...]


_k6_call = pl.pallas_call(
    _k6_body,
    grid=(NN // BR,),
    in_specs=[
        pl.BlockSpec((NC, BR, DD), lambda i: (0, i, 0)),
        pl.BlockSpec((BR, DD), lambda i: (i, 0)),
        pl.BlockSpec((BR, 1), lambda i: (i, 0)),
        pl.BlockSpec((1, DD), lambda i: (0, 0)),
    ],
    out_specs=pl.BlockSpec((BR, DD), lambda i: (i, 0)),
    out_shape=jax.ShapeDtypeStruct((NN, DD), jnp.float32),
)


@jax.jit
def kernel(x, edge_index, W1, b1, a1, W2, b2):
    row = edge_index[0].astype(jnp.int32)
    col = edge_index[1].astype(jnp.int32)
    cole = col.reshape(NW, EPW)
    pad = ((0, 0), (0, EPWP - EPW))
    row2 = jnp.pad(row.reshape(NW, EPW), pad).reshape(NW, NCHUNK, CH)
    col2 = jnp.pad(col.reshape(NW, EPW), pad,
                   constant_values=TRASH).reshape(NW, NCHUNK, CH)
    zeros_blk = jnp.zeros((RPT, DD), jnp.float32)
    b1r = b1.reshape(1, DD)
    b2r = b2.reshape(1, DD)
    a1r = a1.reshape(1, 1)

    degp = _deg_call(cole)
    dinv = _kd_call(degp)
    h1p = _k2_call(x, W1, dinv)
    acc1 = _agg_call(h1p, row2, col2, zeros_blk)
    h2p = _k4_call(acc1, h1p, dinv, W2, b1r, a1r)
    acc2 = _agg_call(h2p, row2, col2, zeros_blk)
    out = _k6_call(acc2, h2p, dinv, b2r)
    return out


# speedup vs baseline: 11.7231x; 1.0736x over previous
import functools
import jax
import jax.numpy as jnp
from jax import lax
from jax.experimental import pallas as pl
from jax.experimental.pallas import tpu as pltpu
from jax.experimental.pallas import tpu_sc as plsc

NN = 10000
EE = 320000
DD = 128
NC = 2
NS = 16
NW = NC * NS
EPW = EE // NW
CH = 64
EPWP = 10240
NCHUNK = EPWP // CH
GRP = 16
NGRP = NCHUNK // GRP
NBUF = 4
NNP = 10240
RPT = NNP // NS
TRASH = 10100

_mesh = plsc.VectorSubcoreMesh(
    core_axis_name="c", subcore_axis_name="s", num_cores=NC, num_subcores=NS
)


def _deg_body(col_hbm, degp_hbm, colv, hist, sem):
    c = lax.axis_index("c")
    s = lax.axis_index("s")
    wid = c * NS + s
    pltpu.async_copy(col_hbm.at[wid], colv, sem).wait()
    zeros16 = jnp.zeros((16,), jnp.float32)

    def zero_step(i, carry):
        hist[pl.ds(i * 16, 16)] = zeros16
        return carry

    lax.fori_loop(0, NN // 16, zero_step, 0)
    ones16 = jnp.ones((16,), jnp.float32)

    def acc_step(i, carry):
        idx = colv[pl.ds(i * 16, 16)]
        plsc.addupdate_scatter(hist, [idx], ones16)
        return carry

    lax.fori_loop(0, EPW // 16, acc_step, 0)
    pltpu.sync_copy(hist, degp_hbm.at[wid])


_deg_call = pl.kernel(
    _deg_body,
    out_type=jax.ShapeDtypeStruct((NW, NN), jnp.float32),
    mesh=_mesh,
    compiler_params=pltpu.CompilerParams(needs_layout_passes=False),
    scratch_types=[
        pltpu.VMEM((EPW,), jnp.int32),
        pltpu.VMEM((NN,), jnp.float32),
        pltpu.SemaphoreType.DMA,
    ],
)


def _agg_body(hp_hbm, row3_hbm, col3_hbm, zeros_hbm, acc_hbm,
              ir0, ic0, ir1, ic1, g0, g1, g2, g3, acc_sh, semi, semg, sems):
    c = lax.axis_index("c")
    s = lax.axis_index("s")
    wid = c * NS + s
    pltpu.sync_copy(zeros_hbm, acc_sh.at[pl.ds(s * RPT, RPT)])
    plsc.subcore_barrier()

    irows = (ir0, ir1)
    icols = (ic0, ic1)
    gbufs = (g0, g1, g2, g3)

    def slab(kind, gi, start=False):
        bufs = irows if kind == 0 else icols
        src = (row3_hbm if kind == 0 else col3_hbm).at[wid].at[pl.ds(gi * GRP, GRP)]
        if start:
            pltpu.async_copy(src, bufs[gi % 2], semi)
        else:
            pltpu.make_async_copy(src, bufs[gi % 2], semi).wait()

    def gather(t, start=False):
        gi, b = divmod(t, GRP)
        src = hp_hbm.at[irows[gi % 2].at[b]]
        if start:
            pltpu.async_copy(src, gbufs[t % NBUF], semg)
        else:
            pltpu.make_async_copy(src, gbufs[t % NBUF], semg).wait()

    def scatter(t, start=False):
        gi, b = divmod(t, GRP)
        dst = acc_sh.at[icols[gi % 2].at[b]]
        if start:
            pltpu.async_copy(gbufs[t % NBUF], dst, sems, add=True)
        else:
            pltpu.make_async_copy(gbufs[t % NBUF], dst, sems).wait()

    slab(0, 0, start=True)
    slab(1, 0, start=True)
    slab(0, 0)
    slab(1, 0)
    for t in range(NBUF - 1):
        gather(t, start=True)
    for t in range(NCHUNK):
        gi, b = divmod(t, GRP)
        gather(t)
        scatter(t, start=True)
        if t >= 1:
            scatter(t - 1)
        if b == 0 and gi + 1 < NGRP:
            slab(0, gi + 1, start=True)
            slab(1, gi + 1, start=True)
        tn = t + NBUF - 1
        if tn < NCHUNK:
            gn, bn = divmod(tn, GRP)
            if bn == 0:
                slab(0, gn)
                slab(1, gn)
            gather(tn, start=True)
    scatter(NCHUNK - 1)

    plsc.subcore_barrier()
    pltpu.sync_copy(
        acc_sh.at[pl.ds(s * RPT, RPT)],
        acc_hbm.at[c].at[pl.ds(s * RPT, RPT)],
    )


_agg_call = pl.kernel(
    _agg_body,
    out_type=jax.ShapeDtypeStruct((NC, NNP, DD), jnp.float32),
    mesh=_mesh,
    scratch_types=[
        pltpu.VMEM((GRP, CH), jnp.int32),
        pltpu.VMEM((GRP, CH), jnp.int32),
        pltpu.VMEM((GRP, CH), jnp.int32),
        pltpu.VMEM((GRP, CH), jnp.int32),
        pltpu.VMEM((CH, DD), jnp.float32),
        pltpu.VMEM((CH, DD), jnp.float32),
        pltpu.VMEM((CH, DD), jnp.float32),
        pltpu.VMEM((CH, DD), jnp.float32),
        pltpu.VMEM_SHARED((NNP, DD), jnp.float32),
        pltpu.SemaphoreType.DMA,
        pltpu.SemaphoreType.DMA,
        pltpu.SemaphoreType.DMA,
    ],
)


BR = 2000


def _kd_body(degp_ref, out_ref):
    deg = jnp.sum(degp_ref[...], axis=0) + 1.0
    out_ref[...] = lax.rsqrt(deg)[:, None]


_kd_call = pl.pallas_call(
    _kd_body,
    in_specs=[pl.BlockSpec((NW, NN), lambda: (0, 0))],
    out_specs=pl.BlockSpec((NN, 1), lambda: (0, 0)),
    out_shape=jax.ShapeDtypeStruct((NN, 1), jnp.float32),
)


def _k2_body(x_ref, w_ref, dinv_ref, out_ref):
    h = jnp.dot(x_ref[...], w_ref[...], preferred_element_type=jnp.float32,
                precision=lax.Precision.HIGHEST)
    out_ref[...] = h * dinv_ref[...]


_k2_call = pl.pallas_call(
    _k2_body,
    grid=(NN // BR,),
    in_specs=[
        pl.BlockSpec((BR, DD), lambda i: (i, 0)),
        pl.BlockSpec((DD, DD), lambda i: (0, 0)),
        pl.BlockSpec((BR, 1), lambda i: (i, 0)),
    ],
    out_specs=pl.BlockSpec((BR, DD), lambda i: (i, 0)),
    out_shape=jax.ShapeDtypeStruct((NN, DD), jnp.float32),
)


def _k4_body(acc_ref, hp_ref, dinv_ref, w_ref, b_ref, a_ref, out_ref):
    dinv = dinv_ref[...]
    tot = acc_ref[0] + acc_ref[1] + hp_ref[...]
    h = tot * dinv + b_ref[...]
    h = jnp.where(h >= 0, h, a_ref[0, 0] * h)
    out_ref[...] = jnp.dot(h, w_ref[...], preferred_element_type=jnp.float32,
                           precision=lax.Precision.HIGHEST) * dinv


_k4_call = pl.pallas_call(
    _k4_body,
    grid=(NN // BR,),
    in_specs=[
        pl.BlockSpec((NC, BR, DD), lambda i: (0, i, 0)),
        pl.BlockSpec((BR, DD), lambda i: (i, 0)),
        pl.BlockSpec((BR, 1), lambda i: (i, 0)),
        pl.BlockSpec((DD, DD), lambda i: (0, 0)),
        pl.BlockSpec((1, DD), lambda i: (0, 0)),
        pl.BlockSpec((1, 1), lambda i: (0, 0), memory_space=pltpu.SMEM),
    ],
    out_specs=pl.BlockSpec((BR, DD), lambda i: (i, 0)),
    out_shape=jax.ShapeDtypeStruct((NN, DD), jnp.float32),
)


def _k6_body(acc_ref, hp_ref, dinv_ref, b_ref, out_ref):
    tot = acc_ref[0] + acc_ref[1] + hp_ref[...]
    out_ref[...] = tot * dinv_ref[...] + b_ref[...]


_k6_call = pl.pallas_call(
    _k6_body,
    grid=(NN // BR,),
    in_specs=[
        pl.BlockSpec((NC, BR, DD), lambda i: (0, i, 0)),
        pl.BlockSpec((BR, DD), lambda i: (i, 0)),
        pl.BlockSpec((BR, 1), lambda i: (i, 0)),
        pl.BlockSpec((1, DD), lambda i: (0, 0)),
    ],
    out_specs=pl.BlockSpec((BR, DD), lambda i: (i, 0)),
    out_shape=jax.ShapeDtypeStruct((NN, DD), jnp.float32),
)


@jax.jit
def kernel(x, edge_index, W1, b1, a1, W2, b2):
    row = edge_index[0].astype(jnp.int32)
    col = edge_index[1].astype(jnp.int32)
    cole = col.reshape(NW, EPW)
    pad = ((0, 0), (0, EPWP - EPW))
    row2 = jnp.pad(row.reshape(NW, EPW), pad).reshape(NW, NCHUNK, CH)
    col2 = jnp.pad(col.reshape(NW, EPW), pad,
                   constant_values=TRASH).reshape(NW, NCHUNK, CH)
    zeros_blk = jnp.zeros((RPT, DD), jnp.float32)
    b1r = b1.reshape(1, DD)
    b2r = b2.reshape(1, DD)
    a1r = a1.reshape(1, 1)

    degp = _deg_call(cole)
    dinv = _kd_call(degp)
    h1p = _k2_call(x, W1, dinv)
    acc1 = _agg_call(h1p, row2, col2, zeros_blk)
    h2p = _k4_call(acc1, h1p, dinv, W2, b1r, a1r)
    acc2 = _agg_call(h2p, row2, col2, zeros_blk)
    out = _k6_call(acc2, h2p, dinv, b2r)
    return out


# speedup vs baseline: 11.7492x; 1.0022x over previous
import functools
import jax
import jax.numpy as jnp
from jax import lax
from jax.experimental import pallas as pl
from jax.experimental.pallas import tpu as pltpu
from jax.experimental.pallas import tpu_sc as plsc

NN = 10000
EE = 320000
DD = 128
NC = 2
NS = 16
NW = NC * NS
EPW = EE // NW
CH = 64
EPWP = 10240
NCHUNK = EPWP // CH
GRP = 8
NGRP = NCHUNK // GRP
NBUF = 5
NNP = 10240
RPT = NNP // NS
TRASH = 10100

_mesh = plsc.VectorSubcoreMesh(
    core_axis_name="c", subcore_axis_name="s", num_cores=NC, num_subcores=NS
)


def _deg_body(col_hbm, degp_hbm, colv, hist, sem):
    c = lax.axis_index("c")
    s = lax.axis_index("s")
    wid = c * NS + s
    pltpu.async_copy(col_hbm.at[wid], colv, sem).wait()
    zeros16 = jnp.zeros((16,), jnp.float32)

    def zero_step(i, carry):
        hist[pl.ds(i * 16, 16)] = zeros16
        return carry

    lax.fori_loop(0, NN // 16, zero_step, 0)
    ones16 = jnp.ones((16,), jnp.float32)

    def acc_step(i, carry):
        idx = colv[pl.ds(i * 16, 16)]
        plsc.addupdate_scatter(hist, [idx], ones16)
        return carry

    lax.fori_loop(0, EPW // 16, acc_step, 0)
    pltpu.sync_copy(hist, degp_hbm.at[wid])


_deg_call = pl.kernel(
    _deg_body,
    out_type=jax.ShapeDtypeStruct((NW, NN), jnp.float32),
    mesh=_mesh,
    compiler_params=pltpu.CompilerParams(needs_layout_passes=False),
    scratch_types=[
        pltpu.VMEM((EPW,), jnp.int32),
        pltpu.VMEM((NN,), jnp.float32),
        pltpu.SemaphoreType.DMA,
    ],
)


def _agg_body(hp_hbm, row3_hbm, col3_hbm, zeros_hbm, acc_hbm,
              ir0, ic0, ir1, ic1, g0, g1, g2, g3, g4, acc_sh, semi, semg, sems):
    c = lax.axis_index("c")
    s = lax.axis_index("s")
    wid = c * NS + s
    pltpu.sync_copy(zeros_hbm, acc_sh.at[pl.ds(s * RPT, RPT)])
    plsc.subcore_barrier()

    irows = (ir0, ir1)
    icols = (ic0, ic1)
    gbufs = (g0, g1, g2, g3, g4)

    def slab(kind, gi, start=False):
        bufs = irows if kind == 0 else icols
        src = (row3_hbm if kind == 0 else col3_hbm).at[wid].at[pl.ds(gi * GRP, GRP)]
        if start:
            pltpu.async_copy(src, bufs[gi % 2], semi)
        else:
            pltpu.make_async_copy(src, bufs[gi % 2], semi).wait()

    def gather(t, start=False):
        gi, b = divmod(t, GRP)
        src = hp_hbm.at[irows[gi % 2].at[b]]
        if start:
            pltpu.async_copy(src, gbufs[t % NBUF], semg)
        else:
            pltpu.make_async_copy(src, gbufs[t % NBUF], semg).wait()

    def scatter(t, start=False):
        gi, b = divmod(t, GRP)
        dst = acc_sh.at[icols[gi % 2].at[b]]
        if start:
            pltpu.async_copy(gbufs[t % NBUF], dst, sems, add=True)
        else:
            pltpu.make_async_copy(gbufs[t % NBUF], dst, sems).wait()

    slab(0, 0, start=True)
    slab(1, 0, start=True)
    slab(0, 0)
    slab(1, 0)
    for t in range(NBUF - 1):
        gather(t, start=True)
    for t in range(NCHUNK):
        gi, b = divmod(t, GRP)
        gather(t)
        scatter(t, start=True)
        if t >= 1:
            scatter(t - 1)
        if b == 0 and gi + 1 < NGRP:
            slab(0, gi + 1, start=True)
            slab(1, gi + 1, start=True)
        tn = t + NBUF - 1
        if tn < NCHUNK:
            gn, bn = divmod(tn, GRP)
            if bn == 0:
                slab(0, gn)
                slab(1, gn)
            gather(tn, start=True)
    scatter(NCHUNK - 1)

    plsc.subcore_barrier()
    pltpu.sync_copy(
        acc_sh.at[pl.ds(s * RPT, RPT)],
        acc_hbm.at[c].at[pl.ds(s * RPT, RPT)],
    )


_agg_call = pl.kernel(
    _agg_body,
    out_type=jax.ShapeDtypeStruct((NC, NNP, DD), jnp.float32),
    mesh=_mesh,
    scratch_types=[
        pltpu.VMEM((GRP, CH), jnp.int32),
        pltpu.VMEM((GRP, CH), jnp.int32),
        pltpu.VMEM((GRP, CH), jnp.int32),
        pltpu.VMEM((GRP, CH), jnp.int32),
        pltpu.VMEM((CH, DD), jnp.float32),
        pltpu.VMEM((CH, DD), jnp.float32),
        pltpu.VMEM((CH, DD), jnp.float32),
        pltpu.VMEM((CH, DD), jnp.float32),
        pltpu.VMEM((CH, DD), jnp.float32),
        pltpu.VMEM_SHARED((NNP, DD), jnp.float32),
        pltpu.SemaphoreType.DMA,
        pltpu.SemaphoreType.DMA,
        pltpu.SemaphoreType.DMA,
    ],
)


BR = 2000


def _kd_body(degp_ref, out_ref):
    deg = jnp.sum(degp_ref[...], axis=0) + 1.0
    out_ref[...] = lax.rsqrt(deg)[:, None]


_kd_call = pl.pallas_call(
    _kd_body,
    in_specs=[pl.BlockSpec((NW, NN), lambda: (0, 0))],
    out_specs=pl.BlockSpec((NN, 1), lambda: (0, 0)),
    out_shape=jax.ShapeDtypeStruct((NN, 1), jnp.float32),
)


def _k2_body(x_ref, w_ref, dinv_ref, out_ref):
    h = jnp.dot(x_ref[...], w_ref[...], preferred_element_type=jnp.float32,
                precision=lax.Precision.HIGHEST)
    out_ref[...] = h * dinv_ref[...]


_k2_call = pl.pallas_call(
    _k2_body,
    grid=(NN // BR,),
    in_specs=[
        pl.BlockSpec((BR, DD), lambda i: (i, 0)),
        pl.BlockSpec((DD, DD), lambda i: (0, 0)),
        pl.BlockSpec((BR, 1), lambda i: (i, 0)),
    ],
    out_specs=pl.BlockSpec((BR, DD), lambda i: (i, 0)),
    out_shape=jax.ShapeDtypeStruct((NN, DD), jnp.float32),
)


def _k4_body(acc_ref, hp_ref, dinv_ref, w_ref, b_ref, a_ref, out_ref):
    dinv = dinv_ref[...]
    tot = acc_ref[0] + acc_ref[1] + hp_ref[...]
    h = tot * dinv + b_ref[...]
    h = jnp.where(h >= 0, h, a_ref[0, 0] * h)
    out_ref[...] = jnp.dot(h, w_ref[...], preferred_element_type=jnp.float32,
                           precision=lax.Precision.HIGHEST) * dinv


_k4_call = pl.pallas_call(
    _k4_body,
    grid=(NN // BR,),
    in_specs=[
        pl.BlockSpec((NC, BR, DD), lambda i: (0, i, 0)),
        pl.BlockSpec((BR, DD), lambda i: (i, 0)),
        pl.BlockSpec((BR, 1), lambda i: (i, 0)),
        pl.BlockSpec((DD, DD), lambda i: (0, 0)),
        pl.BlockSpec((1, DD), lambda i: (0, 0)),
        pl.BlockSpec((1, 1), lambda i: (0, 0), memory_space=pltpu.SMEM),
    ],
    out_specs=pl.BlockSpec((BR, DD), lambda i: (i, 0)),
    out_shape=jax.ShapeDtypeStruct((NN, DD), jnp.float32),
)


def _k6_body(acc_ref, hp_ref, dinv_ref, b_ref, out_ref):
    tot = acc_ref[0] + acc_ref[1] + hp_ref[...]
    out_ref[...] = tot * dinv_ref[...] + b_ref[...]


_k6_call = pl.pallas_call(
    _k6_body,
    grid=(NN // BR,),
    in_specs=[
        pl.BlockSpec((NC, BR, DD), lambda i: (0, i, 0)),
        pl.BlockSpec((BR, DD), lambda i: (i, 0)),
        pl.BlockSpec((BR, 1), lambda i: (i, 0)),
        pl.BlockSpec((1, DD), lambda i: (0, 0)),
    ],
    out_specs=pl.BlockSpec((BR, DD), lambda i: (i, 0)),
    out_shape=jax.ShapeDtypeStruct((NN, DD), jnp.float32),
)


@jax.jit
def kernel(x, edge_index, W1, b1, a1, W2, b2):
    row = edge_index[0].astype(jnp.int32)
    col = edge_index[1].astype(jnp.int32)
    cole = col.reshape(NW, EPW)
    pad = ((0, 0), (0, EPWP - EPW))
    row2 = jnp.pad(row.reshape(NW, EPW), pad).reshape(NW, NCHUNK, CH)
    col2 = jnp.pad(col.reshape(NW, EPW), pad,
                   constant_values=TRASH).reshape(NW, NCHUNK, CH)
    zeros_blk = jnp.zeros((RPT, DD), jnp.float32)
    b1r = b1.reshape(1, DD)
    b2r = b2.reshape(1, DD)
    a1r = a1.reshape(1, 1)

    degp = _deg_call(cole)
    dinv = _kd_call(degp)
    h1p = _k2_call(x, W1, dinv)
    acc1 = _agg_call(h1p, row2, col2, zeros_blk)
    h2p = _k4_call(acc1, h1p, dinv, W2, b1r, a1r)
    acc2 = _agg_call(h2p, row2, col2, zeros_blk)
    out = _k6_call(acc2, h2p, dinv, b2r)
    return out


# speedup vs baseline: 11.7564x; 1.0006x over previous
import jax
import jax.numpy as jnp
from jax import lax
from jax.experimental import pallas as pl
from jax.experimental.pallas import tpu as pltpu
from jax.experimental.pallas import tpu_sc as plsc

NN = 10000
EE = 320000
DD = 128
NC = 2
NS = 16
NW = NC * NS
EPW = EE // NW
CH = 64
EPWP = 10240
NCHUNK = EPWP // CH
GRP = 8
NGRP = NCHUNK // GRP
NBUF = 5
NNP = 10240
RPT = NNP // NS
TRASH = 10100

_mesh = plsc.VectorSubcoreMesh(
    core_axis_name="c", subcore_axis_name="s", num_cores=NC, num_subcores=NS
)


def _deg_body(col_hbm, degp_hbm, colv, hist, sem):
    c = lax.axis_index("c")
    s = lax.axis_index("s")
    wid = c * NS + s
    pltpu.async_copy(col_hbm.at[wid], colv, sem).wait()
    zeros16 = jnp.zeros((16,), jnp.float32)

    def zero_step(i, carry):
        hist[pl.ds(i * 16, 16)] = zeros16
        return carry

    lax.fori_loop(0, NN // 16, zero_step, 0)
    ones16 = jnp.ones((16,), jnp.float32)

    def acc_step(i, carry):
        idx = colv[pl.ds(i * 16, 16)]
        plsc.addupdate_scatter(hist, [idx], ones16)
        return carry

    lax.fori_loop(0, EPW // 16, acc_step, 0)
    pltpu.sync_copy(hist, degp_hbm.at[wid])


_deg_call = pl.kernel(
    _deg_body,
    out_type=jax.ShapeDtypeStruct((NW, NN), jnp.float32),
    mesh=_mesh,
    compiler_params=pltpu.CompilerParams(needs_layout_passes=False),
    scratch_types=[
        pltpu.VMEM((EPW,), jnp.int32),
        pltpu.VMEM((NN,), jnp.float32),
        pltpu.SemaphoreType.DMA,
    ],
)


def _agg_body(hp_hbm, row3_hbm, col3_hbm, zeros_hbm, acc_hbm,
              ir0, ic0, ir1, ic1, g0, g1, g2, g3, g4, acc_sh, semi, semg, sems):
    c = lax.axis_index("c")
    s = lax.axis_index("s")
    wid = c * NS + s
    pltpu.sync_copy(zeros_hbm, acc_sh.at[pl.ds(s * RPT, RPT)])
    plsc.subcore_barrier()

    irows = (ir0, ir1)
    icols = (ic0, ic1)
    gbufs = (g0, g1, g2, g3, g4)

    def slab(kind, gi, start=False):
        bufs = irows if kind == 0 else icols
        src = (row3_hbm if kind == 0 else col3_hbm).at[wid].at[pl.ds(gi * GRP, GRP)]
        if start:
            pltpu.async_copy(src, bufs[gi % 2], semi)
        else:
            pltpu.make_async_copy(src, bufs[gi % 2], semi).wait()

    def gather(t, start=False):
        gi, b = divmod(t, GRP)
        src = hp_hbm.at[irows[gi % 2].at[b]]
        if start:
            pltpu.async_copy(src, gbufs[t % NBUF], semg)
        else:
            pltpu.make_async_copy(src, gbufs[t % NBUF], semg).wait()

    def scatter(t, start=False):
        gi, b = divmod(t, GRP)
        dst = acc_sh.at[icols[gi % 2].at[b]]
        if start:
            pltpu.async_copy(gbufs[t % NBUF], dst, sems, add=True)
        else:
            pltpu.make_async_copy(gbufs[t % NBUF], dst, sems).wait()

    slab(0, 0, start=True)
    slab(1, 0, start=True)
    slab(0, 0)
    slab(1, 0)
    for t in range(NBUF - 1):
        gather(t, start=True)
    for t in range(NCHUNK):
        gi, b = divmod(t, GRP)
        gather(t)
        scatter(t, start=True)
        if t >= 1:
            scatter(t - 1)
        if b == 0 and gi + 1 < NGRP:
            slab(0, gi + 1, start=True)
            slab(1, gi + 1, start=True)
        tn = t + NBUF - 1
        if tn < NCHUNK:
            gn, bn = divmod(tn, GRP)
            if bn == 0:
                slab(0, gn)
                slab(1, gn)
            gather(tn, start=True)
    scatter(NCHUNK - 1)

    plsc.subcore_barrier()
    pltpu.sync_copy(
        acc_sh.at[pl.ds(s * RPT, RPT)],
        acc_hbm.at[c].at[pl.ds(s * RPT, RPT)],
    )


_agg_call = pl.kernel(
    _agg_body,
    out_type=jax.ShapeDtypeStruct((NC, NNP, DD), jnp.float32),
    mesh=_mesh,
    scratch_types=[
        pltpu.VMEM((GRP, CH), jnp.int32),
        pltpu.VMEM((GRP, CH), jnp.int32),
        pltpu.VMEM((GRP, CH), jnp.int32),
        pltpu.VMEM((GRP, CH), jnp.int32),
        pltpu.VMEM((CH, DD), jnp.float32),
        pltpu.VMEM((CH, DD), jnp.float32),
        pltpu.VMEM((CH, DD), jnp.float32),
        pltpu.VMEM((CH, DD), jnp.float32),
        pltpu.VMEM((CH, DD), jnp.float32),
        pltpu.VMEM_SHARED((NNP, DD), jnp.float32),
        pltpu.SemaphoreType.DMA,
        pltpu.SemaphoreType.DMA,
        pltpu.SemaphoreType.DMA,
    ],
)


BR = 2000


def _kd_body(degp_ref, out_ref):
    deg = jnp.sum(degp_ref[...], axis=0) + 1.0
    out_ref[...] = lax.rsqrt(deg)[:, None]


_kd_call = pl.pallas_call(
    _kd_body,
    in_specs=[pl.BlockSpec((NW, NN), lambda: (0, 0))],
    out_specs=pl.BlockSpec((NN, 1), lambda: (0, 0)),
    out_shape=jax.ShapeDtypeStruct((NN, 1), jnp.float32),
)


def _k2_body(x_ref, w_ref, dinv_ref, out_ref):
    h = jnp.dot(x_ref[...], w_ref[...], preferred_element_type=jnp.float32,
                precision=lax.Precision.HIGHEST)
    out_ref[...] = h * dinv_ref[...]


_k2_call = pl.pallas_call(
    _k2_body,
    grid=(NN // BR,),
    in_specs=[
        pl.BlockSpec((BR, DD), lambda i: (i, 0)),
        pl.BlockSpec((DD, DD), lambda i: (0, 0)),
        pl.BlockSpec((BR, 1), lambda i: (i, 0)),
    ],
    out_specs=pl.BlockSpec((BR, DD), lambda i: (i, 0)),
    out_shape=jax.ShapeDtypeStruct((NN, DD), jnp.float32),
)


def _k4_body(acc_ref, hp_ref, dinv_ref, w_ref, b_ref, a_ref, out_ref):
    dinv = dinv_ref[...]
    tot = acc_ref[0] + acc_ref[1] + hp_ref[...]
    h = tot * dinv + b_ref[...]
    h = jnp.where(h >= 0, h, a_ref[0, 0] * h)
    out_ref[...] = jnp.dot(h, w_ref[...], preferred_element_type=jnp.float32,
                           precision=lax.Precision.HIGHEST) * dinv


_k4_call = pl.pallas_call(
    _k4_body,
    grid=(NN // BR,),
    in_specs=[
        pl.BlockSpec((NC, BR, DD), lambda i: (0, i, 0)),
        pl.BlockSpec((BR, DD), lambda i: (i, 0)),
        pl.BlockSpec((BR, 1), lambda i: (i, 0)),
        pl.BlockSpec((DD, DD), lambda i: (0, 0)),
        pl.BlockSpec((1, DD), lambda i: (0, 0)),
        pl.BlockSpec((1, 1), lambda i: (0, 0), memory_space=pltpu.SMEM),
    ],
    out_specs=pl.BlockSpec((BR, DD), lambda i: (i, 0)),
    out_shape=jax.ShapeDtypeStruct((NN, DD), jnp.float32),
)


def _k6_body(acc_ref, hp_ref, dinv_ref, b_ref, out_ref):
    tot = acc_ref[0] + acc_ref[1] + hp_ref[...]
    out_ref[...] = tot * dinv_ref[...] + b_ref[...]


_k6_call = pl.pallas_call(
    _k6_body,
    grid=(NN // BR,),
    in_specs=[
        pl.BlockSpec((NC, BR, DD), lambda i: (0, i, 0)),
        pl.BlockSpec((BR, DD), lambda i: (i, 0)),
        pl.BlockSpec((BR, 1), lambda i: (i, 0)),
        pl.BlockSpec((1, DD), lambda i: (0, 0)),
    ],
    out_specs=pl.BlockSpec((BR, DD), lambda i: (i, 0)),
    out_shape=jax.ShapeDtypeStruct((NN, DD), jnp.float32),
)


@jax.jit
def kernel(x, edge_index, W1, b1, a1, W2, b2):
    row = edge_index[0].astype(jnp.int32)
    col = edge_index[1].astype(jnp.int32)
    cole = col.reshape(NW, EPW)
    pad = ((0, 0), (0, EPWP - EPW))
    row2 = jnp.pad(row.reshape(NW, EPW), pad).reshape(NW, NCHUNK, CH)
    col2 = jnp.pad(col.reshape(NW, EPW), pad,
                   constant_values=TRASH).reshape(NW, NCHUNK, CH)
    zeros_blk = jnp.zeros((RPT, DD), jnp.float32)
    b1r = b1.reshape(1, DD)
    b2r = b2.reshape(1, DD)
    a1r = a1.reshape(1, 1)

    degp = _deg_call(cole)
    dinv = _kd_call(degp)
    h1p = _k2_call(x, W1, dinv)
    acc1 = _agg_call(h1p, row2, col2, zeros_blk)
    h2p = _k4_call(acc1, h1p, dinv, W2, b1r, a1r)
    acc2 = _agg_call(h2p, row2, col2, zeros_blk)
    out = _k6_call(acc2, h2p, dinv, b2r)
    return out
